# baseline (speedup 1.0000x reference)
import jax
import jax.numpy as jnp
from jax import lax
from jax.experimental import pallas as pl
from jax.experimental.pallas import tpu as pltpu
from jax.experimental.pallas import tpu_sc as plsc

B = 4096
Q = 100
D = 64
DP = 128
L = 16
NC = 2
NS = 16
NW = NC * NS
BPW = B // NW
NBUF_G = 4
NBUF_O = 2
C = D // L


def kernel(answer, year, answer_table, yearly_table, question_table):
    answer_t = answer.astype(jnp.int32).T
    year_flat = year.reshape(-1).astype(jnp.int32)
    atab_pad = jnp.pad(answer_table, ((0, 0), (0, DP - D)))
    ytab_pad = jnp.pad(yearly_table, ((0, 0), (0, DP - D)))
    qtab_pad = jnp.pad(question_table, ((0, 0), (0, DP - D)))

    def body(answer_hbm, year_hbm, atab_hbm, ytab_hbm, qtab_hbm, out_hbm,
             idx_v, yidx_v, yr_v, qt_v, rows_v, stage_v,
             gsem, osem, ysem):
        wid = lax.axis_index("s") * NC + lax.axis_index("c")
        base = wid * BPW

        pltpu.sync_copy(answer_hbm.at[:, pl.ds(base, BPW)], idx_v)
        pltpu.sync_copy(year_hbm.at[pl.ds(base, BPW)], yidx_v)
        pltpu.sync_copy(qtab_hbm, qt_v)
        pltpu.async_copy(ytab_hbm.at[yidx_v], yr_v, ysem).wait()

        lane = lax.iota(jnp.int32, L)
        dsel = tuple(lane + (c * L) for c in range(C))

        for j in range(NBUF_G):
            pltpu.async_copy(atab_hbm.at[idx_v.at[j]], rows_v.at[j],
                             gsem.at[j])

        def outer(o, carry):
            for j in range(NBUF_G):
                q = o * NBUF_G + j
                oj = j % NBUF_O
                pltpu.make_async_copy(
                    atab_hbm.at[idx_v.at[q]], rows_v.at[j], gsem.at[j]).wait()

                def _wait_out():
                    pltpu.make_async_copy(
                        stage_v.at[oj], out_hbm.at[0, :, pl.ds(base, BPW)],
                        osem.at[oj]).wait()
                if j >= NBUF_O:
                    _wait_out()
                else:
                    pl.when(o > 0)(_wait_out)

                qrow = tuple(qt_v[q, pl.ds(c * L, L)] for c in range(C))

                def bbody(b, ycarry):
                    bvec = jnp.full((L,), b, jnp.int32)
                    for c in range(C):
                        s = (rows_v[j, b, pl.ds(c * L, L)]
                             + yr_v[b, pl.ds(c * L, L)]
                             + ycarry[c])
                        plsc.store_scatter(stage_v.at[oj], [dsel[c], bvec], s)
                    return ycarry

                lax.fori_loop(0, BPW, bbody, qrow)

                pltpu.async_copy(stage_v.at[oj],
                                 out_hbm.at[q, :, pl.ds(base, BPW)],
                                 osem.at[oj])

                def _next_gather():
                    pltpu.async_copy(
                        atab_hbm.at[idx_v.at[q + NBUF_G]], rows_v.at[j],
                        gsem.at[j])
                pl.when(o < Q // NBUF_G - 1)(_next_gather)
            return carry

        lax.fori_loop(0, Q // NBUF_G, outer, 0)

        for oj in range(NBUF_O):
            pltpu.make_async_copy(
                stage_v.at[oj], out_hbm.at[0, :, pl.ds(base, BPW)],
                osem.at[oj]).wait()

    mesh = plsc.VectorSubcoreMesh(core_axis_name="c", subcore_axis_name="s",
                                  num_cores=NC, num_subcores=NS)
    run = pl.kernel(
        body,
        out_type=jax.ShapeDtypeStruct((Q, D, B), jnp.float32),
        mesh=mesh,
        compiler_params=pltpu.CompilerParams(needs_layout_passes=False),
        scratch_types=[
            pltpu.VMEM((Q, BPW), jnp.int32),
            pltpu.VMEM((BPW,), jnp.int32),
            pltpu.VMEM((BPW, DP), jnp.float32),
            pltpu.VMEM((Q, DP), jnp.float32),
            pltpu.VMEM((NBUF_G, BPW, DP), jnp.float32),
            pltpu.VMEM((NBUF_O, D, BPW), jnp.float32),
            pltpu.SemaphoreType.DMA((NBUF_G,)),
            pltpu.SemaphoreType.DMA((NBUF_O,)),
            pltpu.SemaphoreType.DMA,
        ],
    )
    out_t = run(answer_t, year_flat, atab_pad, ytab_pad, qtab_pad)
    return jnp.transpose(out_t, (2, 0, 1))


# speedup vs baseline: 1.5594x; 1.5594x over previous
import jax
import jax.numpy as jnp
from jax import lax
from jax.experimental import pallas as pl
from jax.experimental.pallas import tpu as pltpu
from jax.experimental.pallas import tpu_sc as plsc

B = 4096
Q = 100
D = 64
DP = 128
L = 16
NC = 2
NS = 16
NW = NC * NS
BPW = B // NW
NBUF_G = 4
NBUF_O = 2
C = D // L


def kernel(answer, year, answer_table, yearly_table, question_table):
    answer_t = answer.astype(jnp.int32).T
    year_flat = year.reshape(-1).astype(jnp.int32)
    atab_pad = jnp.pad(answer_table, ((0, 0), (0, DP - D)))
    ytab_pad = jnp.pad(yearly_table, ((0, 0), (0, DP - D)))
    qtab_pad = jnp.pad(question_table, ((0, 0), (0, DP - D)))

    def body(answer_hbm, year_hbm, atab_hbm, ytab_hbm, qtab_hbm, out_hbm,
             idx_v, yidx_v, yr_v, qt_v, rows_v, stage_v,
             gsem, osem, ysem):
        wid = lax.axis_index("s") * NC + lax.axis_index("c")
        base = wid * BPW

        pltpu.sync_copy(answer_hbm.at[:, pl.ds(base, BPW)], idx_v)
        pltpu.sync_copy(year_hbm.at[pl.ds(base, BPW)], yidx_v)
        pltpu.sync_copy(qtab_hbm, qt_v)
        pltpu.async_copy(ytab_hbm.at[yidx_v], yr_v, ysem).wait()

        lane = lax.iota(jnp.int32, L)
        dsel = tuple(lane + (c * L) for c in range(C))

        for j in range(NBUF_G):
            pltpu.async_copy(atab_hbm.at[idx_v.at[j]], rows_v.at[j],
                             gsem.at[j])

        def outer(o, carry):
            for j in range(NBUF_G):
                q = o * NBUF_G + j
                oj = j % NBUF_O
                pltpu.make_async_copy(
                    atab_hbm.at[idx_v.at[q]], rows_v.at[j], gsem.at[j]).wait()

                def _wait_out():
                    pltpu.make_async_copy(
                        stage_v.at[oj], out_hbm.at[0, :, pl.ds(base, BPW)],
                        osem.at[oj]).wait()
                if j >= NBUF_O:
                    _wait_out()
                else:
                    pl.when(o > 0)(_wait_out)

                qrow = tuple(qt_v[q, pl.ds(c * L, L)] for c in range(C))

                @plsc.parallel_loop(0, BPW, step=1, unroll=8, carry=qrow)
                def bbody(b, qc):
                    bvec = jnp.full((L,), b, jnp.int32)
                    for c in range(C):
                        s = (rows_v[j, b, pl.ds(c * L, L)]
                             + yr_v[b, pl.ds(c * L, L)]
                             + qc[c])
                        plsc.store_scatter(stage_v.at[oj], [dsel[c], bvec], s)
                    return qc

                pltpu.async_copy(stage_v.at[oj],
                                 out_hbm.at[q, :, pl.ds(base, BPW)],
                                 osem.at[oj])

                def _next_gather():
                    pltpu.async_copy(
                        atab_hbm.at[idx_v.at[q + NBUF_G]], rows_v.at[j],
                        gsem.at[j])
                pl.when(o < Q // NBUF_G - 1)(_next_gather)
            return carry

        lax.fori_loop(0, Q // NBUF_G, outer, 0)

        for oj in range(NBUF_O):
            pltpu.make_async_copy(
                stage_v.at[oj], out_hbm.at[0, :, pl.ds(base, BPW)],
                osem.at[oj]).wait()

    mesh = plsc.VectorSubcoreMesh(core_axis_name="c", subcore_axis_name="s",
                                  num_cores=NC, num_subcores=NS)
    run = pl.kernel(
        body,
        out_type=jax.ShapeDtypeStruct((Q, D, B), jnp.float32),
        mesh=mesh,
        compiler_params=pltpu.CompilerParams(needs_layout_passes=False),
        scratch_types=[
            pltpu.VMEM((Q, BPW), jnp.int32),
            pltpu.VMEM((BPW,), jnp.int32),
            pltpu.VMEM((BPW, DP), jnp.float32),
            pltpu.VMEM((Q, DP), jnp.float32),
            pltpu.VMEM((NBUF_G, BPW, DP), jnp.float32),
            pltpu.VMEM((NBUF_O, D, BPW), jnp.float32),
            pltpu.SemaphoreType.DMA((NBUF_G,)),
            pltpu.SemaphoreType.DMA((NBUF_O,)),
            pltpu.SemaphoreType.DMA,
        ],
    )
    out_t = run(answer_t, year_flat, atab_pad, ytab_pad, qtab_pad)
    return jnp.transpose(out_t, (2, 0, 1))
